# baseline (speedup 1.0000x reference)
import functools

import jax
import jax.numpy as jnp
from jax import lax
from jax.experimental import pallas as pl
from jax.experimental.pallas import tpu as pltpu
from jax.experimental.pallas import tpu_sc as plsc

F32 = jnp.float32
I32 = jnp.int32


def _sc_agg_body(chunks, rsz, h2_hbm, src2d_hbm, dst2d_hbm, zeros_hbm,
                 z2_hbm, idx_s, idx_d, r0, r1, r2, r3,
                 g0, g1, g2, g3, s0, s1, s2, s3, zsp):
    c = lax.axis_index("c")
    s = lax.axis_index("s")
    wid = s * 2 + c
    rows = (r0, r1, r2, r3)
    semg = (g0, g1, g2, g3)
    sems = (s0, s1, s2, s3)
    half = chunks // 4

    @pl.when(s == 0)
    def _():
        pltpu.sync_copy(zeros_hbm, zsp)

    plsc.subcore_barrier()

    for h in range(4):
        pltpu.sync_copy(
            src2d_hbm.at[pl.ds(wid * chunks + h * half, half)], idx_s)
        pltpu.sync_copy(
            dst2d_hbm.at[pl.ds(wid * chunks + h * half, half)], idx_d)
        pltpu.async_copy(h2_hbm.at[idx_s.at[0]], rows[0], semg[0])
        pltpu.async_copy(h2_hbm.at[idx_s.at[1]], rows[1], semg[1])

        def body(g, carry):
            for b in range(4):
                ch = g * 4 + b
                b2 = (b + 2) % 4

                @pl.when(ch >= 2)
                def _():
                    pltpu.make_async_copy(
                        rows[b2], zsp.at[idx_d.at[ch - 2]], sems[b2]).wait()

                @pl.when(ch < half - 2)
                def _():
                    pltpu.async_copy(h2_hbm.at[idx_s.at[ch + 2]], rows[b2],
                                     semg[b2])

                pltpu.make_async_copy(h2_hbm.at[idx_s.at[ch]], rows[b],
                                      semg[b]).wait()
                pltpu.async_copy(rows[b], zsp.at[idx_d.at[ch]], sems[b],
                                 add=True)
            return carry

        lax.fori_loop(0, half // 4, body, 0)
        pltpu.make_async_copy(rows[(half - 2) % 4],
                              zsp.at[idx_d.at[half - 2]],
                              sems[(half - 2) % 4]).wait()
        pltpu.make_async_copy(rows[(half - 1) % 4],
                              zsp.at[idx_d.at[half - 1]],
                              sems[(half - 1) % 4]).wait()

    plsc.subcore_barrier()

    @pl.when(s == 0)
    def _():
        pltpu.sync_copy(zsp, z2_hbm.at[c])


def _sc_aggregate(h2, src2d, dst2d, zeros_z):
    npad, d = h2.shape
    rsz = 64
    nchunks = src2d.shape[0] // 32
    assert nchunks % 16 == 0
    mesh = plsc.VectorSubcoreMesh(core_axis_name="c", subcore_axis_name="s")
    return pl.kernel(
        functools.partial(_sc_agg_body, nchunks, rsz),
        out_type=jax.ShapeDtypeStruct((2, npad, d), F32),
        mesh=mesh,
        scratch_types=[
            pltpu.VMEM((nchunks // 4, rsz), I32),
            pltpu.VMEM((nchunks // 4, rsz), I32),
        ] + [pltpu.VMEM((rsz, d), F32)] * 4
          + [pltpu.SemaphoreType.DMA] * 8
          + [pltpu.VMEM_SHARED((npad, d), F32)],
    )(h2, src2d, dst2d, zeros_z)


def _sc_passa_body(et, src_hbm, dst_hbm, w_hbm, f3_hbm, zeros_hbm,
                   out_hbm, src_c, dst_c, w_c,
                   f0_v, f1_v, f2_v, df0, df1, df2, gm0, gm1, gm2):
    c = lax.axis_index("c")
    s = lax.axis_index("s")
    wid = s * 2 + c
    base = wid * et
    f_vs = (f0_v, f1_v, f2_v)
    df_vs = (df0, df1, df2)
    gm_vs = (gm0, gm1, gm2)
    npad = f0_v.shape[0]
    for t in range(3):
        pltpu.sync_copy(f3_hbm.at[pl.ds(t * npad, npad)], f_vs[t])
        pltpu.sync_copy(zeros_hbm, df_vs[t])
        pltpu.sync_copy(zeros_hbm, gm_vs[t])

    def outer(o, carry):
        off = base + o * 1024
        pltpu.sync_copy(src_hbm.at[pl.ds(off, 1024)], src_c)
        pltpu.sync_copy(dst_hbm.at[pl.ds(off, 1024)], dst_c)
        pltpu.sync_copy(w_hbm.at[pl.ds(off, 1024)], w_c)

        def inner(k, carry2):
            s16 = src_c[pl.ds(k * 16, 16)]
            d16 = dst_c[pl.ds(k * 16, 16)]
            w16 = w_c[pl.ds(k * 16, 16)]
            for t in range(3):
                fs = plsc.load_gather(f_vs[t], [s16])
                fd = plsc.load_gather(f_vs[t], [d16]) - fs
                wfd = w16 * fd
                plsc.addupdate_scatter(df_vs[t], [s16], wfd)
                plsc.addupdate_scatter(gm_vs[t], [s16], wfd * fd)
            return carry2

        return lax.fori_loop(0, 64, inner, carry)

    lax.fori_loop(0, et // 1024, outer, 0)
    for t in range(3):
        pltpu.sync_copy(df_vs[t], out_hbm.at[pl.ds((wid * 6 + t) * npad, npad)])
        pltpu.sync_copy(gm_vs[t],
                        out_hbm.at[pl.ds((wid * 6 + 3 + t) * npad, npad)])


def _sc_pass_a(srcp, dstp, wp, f3, zeros_n):
    npad = f3.shape[0] // 3
    et = srcp.shape[0] // 32
    mesh = plsc.VectorSubcoreMesh(core_axis_name="c", subcore_axis_name="s")
    return pl.kernel(
        functools.partial(_sc_passa_body, et),
        out_type=jax.ShapeDtypeStruct((32 * 6 * npad,), F32),
        mesh=mesh,
        compiler_params=pltpu.CompilerParams(needs_layout_passes=False),
        scratch_types=[
            pltpu.VMEM((1024,), I32),
            pltpu.VMEM((1024,), I32),
            pltpu.VMEM((1024,), F32),
        ] + [pltpu.VMEM((npad,), F32)] * 9,
    )(srcp, dstp, wp, f3, zeros_n)


def _sc_passb_body(et, src_hbm, dst_hbm, w_hbm, f_hbm, df_hbm, gm_hbm,
                   zeros_hbm, out_hbm,
                   src_c, dst_c, w_c, f_v, df_v, gm_v, dg_v, gf_v):
    c = lax.axis_index("c")
    s = lax.axis_index("s")
    wid = s * 2 + c
    base = wid * et
    pltpu.sync_copy(f_hbm, f_v)
    pltpu.sync_copy(df_hbm, df_v)
    pltpu.sync_copy(gm_hbm, gm_v)
    pltpu.sync_copy(zeros_hbm, dg_v)
    pltpu.sync_copy(zeros_hbm, gf_v)

    def outer(o, carry):
        off = base + o * 1024
        pltpu.sync_copy(src_hbm.at[pl.ds(off, 1024)], src_c)
        pltpu.sync_copy(dst_hbm.at[pl.ds(off, 1024)], dst_c)
        pltpu.sync_copy(w_hbm.at[pl.ds(off, 1024)], w_c)

        def inner(k, carry2):
            s16 = src_c[pl.ds(k * 16, 16)]
            d16 = dst_c[pl.ds(k * 16, 16)]
            w16 = w_c[pl.ds(k * 16, 16)]
            fd = plsc.load_gather(f_v, [d16]) - plsc.load_gather(f_v, [s16])
            gd = plsc.load_gather(gm_v, [d16]) - plsc.load_gather(gm_v, [s16])
            dfd = plsc.load_gather(df_v, [d16]) - plsc.load_gather(df_v, [s16])
            plsc.addupdate_scatter(dg_v, [s16], w16 * gd)
            plsc.addupdate_scatter(gf_v, [s16], w16 * fd * dfd)
            return carry2

        return lax.fori_loop(0, 64, inner, carry)

    lax.fori_loop(0, et // 1024, outer, 0)
    npad = dg_v.shape[0]
    pltpu.sync_copy(dg_v, out_hbm.at[pl.ds((wid * 2 + 0) * npad, npad)])
    pltpu.sync_copy(gf_v, out_hbm.at[pl.ds((wid * 2 + 1) * npad, npad)])


def _sc_pass_b(srcp, dstp, wp, f, df, gm, zeros_n):
    npad = f.shape[0]
    et = srcp.shape[0] // 32
    mesh = plsc.VectorSubcoreMesh(core_axis_name="c", subcore_axis_name="s")
    return pl.kernel(
        functools.partial(_sc_passb_body, et),
        out_type=jax.ShapeDtypeStruct((32 * 2 * npad,), F32),
        mesh=mesh,
        compiler_params=pltpu.CompilerParams(needs_layout_passes=False),
        scratch_types=[
            pltpu.VMEM((1024,), I32),
            pltpu.VMEM((1024,), I32),
            pltpu.VMEM((1024,), F32),
        ] + [pltpu.VMEM((npad,), F32)] * 5,
    )(srcp, dstp, wp, f, df, gm, zeros_n)


def _sc_deg_body(et, n_real, src_hbm, dst_hbm, r1_hbm, r2_hbm, zeros_hbm,
                 cnt_hbm, dstm1_hbm, dstm2_hbm,
                 src_c, dst_c, m1_c, m2_c, r1_v, r2_v, dg0, dg1, dg2):
    c = lax.axis_index("c")
    s = lax.axis_index("s")
    wid = s * 2 + c
    base = wid * et
    pltpu.sync_copy(r1_hbm, r1_v)
    pltpu.sync_copy(r2_hbm, r2_v)
    pltpu.sync_copy(zeros_hbm, dg0)
    pltpu.sync_copy(zeros_hbm, dg1)
    pltpu.sync_copy(zeros_hbm, dg2)
    ones16 = jnp.ones((16,), F32)
    lane16 = lax.broadcasted_iota(I32, (16,), 0)

    def outer(o, carry):
        off = base + o * 1024
        pltpu.sync_copy(src_hbm.at[pl.ds(off, 1024)], src_c)
        pltpu.sync_copy(dst_hbm.at[pl.ds(off, 1024)], dst_c)

        def inner(k, carry2):
            s16 = src_c[pl.ds(k * 16, 16)]
            d16 = dst_c[pl.ds(k * 16, 16)]
            r1s = plsc.load_gather(r1_v, [s16])
            r1d = plsc.load_gather(r1_v, [d16])
            r2s = plsc.load_gather(r2_v, [s16])
            r2d = plsc.load_gather(r2_v, [d16])
            keep1 = (r1s == 0.0) & (r1d == 0.0)
            keep2 = (r2s == 0.0) & (r2d == 0.0)
            plsc.addupdate_scatter(dg0, [d16], ones16)
            plsc.addupdate_scatter(dg1, [d16], jnp.where(keep1, 1.0, 0.0))
            plsc.addupdate_scatter(dg2, [d16], jnp.where(keep2, 1.0, 0.0))
            trash16 = n_real + ((off + k * 16 + lane16) & 127)
            m1_c[pl.ds(k * 16, 16)] = jnp.where(keep1, d16, trash16)
            m2_c[pl.ds(k * 16, 16)] = jnp.where(keep2, d16, trash16)
            return carry2

        lax.fori_loop(0, 64, inner, carry)
        pltpu.sync_copy(m1_c, dstm1_hbm.at[pl.ds(off, 1024)])
        pltpu.sync_copy(m2_c, dstm2_hbm.at[pl.ds(off, 1024)])
        return carry

    lax.fori_loop(0, et // 1024, outer, 0)
    npad = dg0.shape[0]
    pltpu.sync_copy(dg0, cnt_hbm.at[pl.ds((wid * 3 + 0) * npad, npad)])
    pltpu.sync_copy(dg1, cnt_hbm.at[pl.ds((wid * 3 + 1) * npad, npad)])
    pltpu.sync_copy(dg2, cnt_hbm.at[pl.ds((wid * 3 + 2) * npad, npad)])


def _sc_deg_mask(srcp, dst0, r1, r2, zeros_n, n_real):
    ep = srcp.shape[0]
    et = ep // 32
    npad = r1.shape[0]
    mesh = plsc.VectorSubcoreMesh(core_axis_name="c", subcore_axis_name="s")
    return pl.kernel(
        functools.partial(_sc_deg_body, et, n_real),
        compiler_params=pltpu.CompilerParams(needs_layout_passes=False),
        out_type=[
            jax.ShapeDtypeStruct((32 * 3 * npad,), F32),
            jax.ShapeDtypeStruct((ep,), I32),
            jax.ShapeDtypeStruct((ep,), I32),
        ],
        mesh=mesh,
        scratch_types=[
            pltpu.VMEM((1024,), I32),
            pltpu.VMEM((1024,), I32),
            pltpu.VMEM((1024,), I32),
            pltpu.VMEM((1024,), I32),
            pltpu.VMEM((npad,), F32),
            pltpu.VMEM((npad,), F32),
            pltpu.VMEM((npad,), F32),
            pltpu.VMEM((npad,), F32),
            pltpu.VMEM((npad,), F32),
        ],
    )(srcp, dst0, r1, r2, zeros_n)


def _sumtiles_body(x_ref, o_ref):
    o_ref[...] = jnp.sum(x_ref[...], axis=0, keepdims=True)


def _sum_tiles(x, cb=2048):
    t, m = x.shape
    return pl.pallas_call(
        _sumtiles_body,
        grid=(m // cb,),
        in_specs=[pl.BlockSpec((t, cb), lambda i: (0, i))],
        out_specs=pl.BlockSpec((1, cb), lambda i: (0, i)),
        out_shape=jax.ShapeDtypeStruct((1, m), F32),
    )(x)


def _node_mlp_body(x_ref, w1_ref, b1_ref, w2_ref, b2_ref, o_ref):
    h = jnp.maximum(
        jnp.dot(x_ref[...], w1_ref[...], preferred_element_type=F32) + b1_ref[...],
        0.0)
    o_ref[...] = jax.nn.sigmoid(
        jnp.dot(h, w2_ref[...], preferred_element_type=F32) + b2_ref[...])


def _node_mlps(x_pad, w1cat, b1cat, w2bd, b2cat, nb=1024):
    npad = x_pad.shape[0]
    d = x_pad.shape[1]
    k = w1cat.shape[1]
    grid = (npad // nb,)
    return pl.pallas_call(
        _node_mlp_body,
        grid=grid,
        in_specs=[
            pl.BlockSpec((nb, d), lambda i: (i, 0)),
            pl.BlockSpec((d, k), lambda i: (0, 0)),
            pl.BlockSpec((1, k), lambda i: (0, 0)),
            pl.BlockSpec((k, 4), lambda i: (0, 0)),
            pl.BlockSpec((1, 4), lambda i: (0, 0)),
        ],
        out_specs=pl.BlockSpec((nb, 4), lambda i: (i, 0)),
        out_shape=jax.ShapeDtypeStruct((npad, 4), F32),
    )(x_pad, w1cat, b1cat, w2bd, b2cat)


def _colsum_body(w_ref, o_ref):
    part = jnp.sum(w_ref[...], axis=0, keepdims=True)

    @pl.when(pl.program_id(0) == 0)
    def _():
        o_ref[...] = part

    @pl.when(pl.program_id(0) != 0)
    def _():
        o_ref[...] = o_ref[...] + part


def _colsum(w1, eb=3200):
    e, c = w1.shape
    return pl.pallas_call(
        _colsum_body,
        grid=(e // eb,),
        in_specs=[pl.BlockSpec((eb, c), lambda i: (i, 0))],
        out_specs=pl.BlockSpec((1, c), lambda i: (0, 0)),
        out_shape=jax.ShapeDtypeStruct((1, c), F32),
    )(w1)


def _wmlp_body(cs_ref, b1_ref, w2_ref, b2_ref, w3_ref, b3_ref, o_ref, h2_ref):
    @pl.when(pl.program_id(0) == 0)
    def _():
        h1 = jnp.maximum(cs_ref[...] + b1_ref[...], 0.0)
        h2_ref[...] = jnp.maximum(
            jnp.dot(h1, w2_ref[...], preferred_element_type=F32) + b2_ref[...], 0.0)

    o_ref[...] = jax.nn.sigmoid(
        jnp.dot(h2_ref[...], w3_ref[...], preferred_element_type=F32) + b3_ref[...])


def _edge_weights(cs, wb1, wW2, wb2, wW3, wb3, cb=3200):
    e = wW3.shape[1]
    return pl.pallas_call(
        _wmlp_body,
        grid=(e // cb,),
        in_specs=[
            pl.BlockSpec((1, 64), lambda i: (0, 0)),
            pl.BlockSpec((1, 64), lambda i: (0, 0)),
            pl.BlockSpec((64, 64), lambda i: (0, 0)),
            pl.BlockSpec((1, 64), lambda i: (0, 0)),
            pl.BlockSpec((64, cb), lambda i: (0, i)),
            pl.BlockSpec((1, cb), lambda i: (0, i)),
        ],
        out_specs=pl.BlockSpec((1, cb), lambda i: (0, i)),
        out_shape=jax.ShapeDtypeStruct((1, e), F32),
        scratch_shapes=[pltpu.VMEM((1, 64), F32)],
    )(cs, wb1, wW2, wb2, wW3, wb3)


def _nsq_body(n_real, rb, cb, krow_ref, pack_ref, loss_ref, rank_ref):
    i = pl.program_id(0)
    j = pl.program_id(1)
    krow = krow_ref[...]
    pack = pack_ref[...]
    kcol = pack[0:1, :]
    row_gid = jax.lax.broadcasted_iota(jnp.int32, (rb, 1), 0) + i * rb
    col_gid = jax.lax.broadcasted_iota(jnp.int32, (1, cb), 1) + j * cb
    rowvalid = (row_gid < n_real).astype(F32)

    acc = jnp.maximum(krow * pack[1:2, :] - pack[4:5, :], 0.0)
    acc = acc + jnp.maximum(krow * pack[2:3, :] - pack[5:6, :], 0.0)
    acc = acc + jnp.maximum(krow * pack[3:4, :] - pack[6:7, :], 0.0)
    loss_tile = jnp.sum(jnp.sum(acc, axis=1, keepdims=True) * rowvalid)

    gt = (kcol > krow).astype(F32)
    eqlow = jnp.where((kcol == krow) & (col_gid < row_gid), 1.0, 0.0)
    rblk = jnp.sum(gt + eqlow, axis=1, keepdims=True)

    @pl.when((i == 0) & (j == 0))
    def _():
        loss_ref[...] = jnp.zeros((1, 1), F32)

    @pl.when(j == 0)
    def _():
        loss_ref[...] += jnp.full((1, 1), -3.0) * jnp.sum(krow * rowvalid)
        rank_ref[...] = rblk

    @pl.when(j != 0)
    def _():
        rank_ref[...] = rank_ref[...] + rblk

    loss_ref[...] += loss_tile.reshape(1, 1)


def _nsq_pass(krows, cols_pack, n_real, rb=256, cb=1024):
    npad = krows.shape[0]
    grid = (npad // rb, npad // cb)
    return pl.pallas_call(
        functools.partial(_nsq_body, n_real, rb, cb),
        grid=grid,
        in_specs=[
            pl.BlockSpec((rb, 1), lambda i, j: (i, 0)),
            pl.BlockSpec((8, cb), lambda i, j: (0, j)),
        ],
        out_specs=[
            pl.BlockSpec((1, 1), lambda i, j: (0, 0)),
            pl.BlockSpec((rb, 1), lambda i, j: (i, 0)),
        ],
        out_shape=[
            jax.ShapeDtypeStruct((1, 1), F32),
            jax.ShapeDtypeStruct((npad, 1), F32),
        ],
    )(krows, cols_pack)


def _mm_scale_body(hx_ref, w_ref, cnt_ref, o_ref):
    dis = lax.rsqrt(cnt_ref[...] + 1.0)
    o_ref[...] = dis * jnp.dot(
        hx_ref[...], w_ref[...], preferred_element_type=F32)


def _mm_scale(hx, w, cnt, nb=1024):
    npad, d = hx.shape
    o = w.shape[1]
    return pl.pallas_call(
        _mm_scale_body,
        grid=(npad // nb,),
        in_specs=[
            pl.BlockSpec((nb, d), lambda i: (i, 0)),
            pl.BlockSpec((d, o), lambda i: (0, 0)),
            pl.BlockSpec((nb, 1), lambda i: (i, 0)),
        ],
        out_specs=pl.BlockSpec((nb, o), lambda i: (i, 0)),
        out_shape=jax.ShapeDtypeStruct((npad, o), F32),
    )(hx, w, cnt)


def _epi_body(final, za_ref, zb_ref, h2_ref, cnt_ref, b_ref, o_ref):
    dis = lax.rsqrt(cnt_ref[...] + 1.0)
    t = dis * (za_ref[0] + zb_ref[0] + h2_ref[...]) + b_ref[...]
    if final:
        s = jax.nn.sigmoid(t)
        o_ref[...] = jnp.sum(s, axis=1, keepdims=True) * (1.0 / s.shape[1])
    else:
        o_ref[...] = jnp.maximum(t, 0.0)


def _epilogue(z2, h2, cnt, b, final, nb=1024):
    _, npad, d = z2.shape
    oc = 1 if final else d
    return pl.pallas_call(
        functools.partial(_epi_body, final),
        grid=(npad // nb,),
        in_specs=[
            pl.BlockSpec((1, nb, d), lambda i: (0, i, 0)),
            pl.BlockSpec((1, nb, d), lambda i: (1, i, 0)),
            pl.BlockSpec((nb, d), lambda i: (i, 0)),
            pl.BlockSpec((nb, 1), lambda i: (i, 0)),
            pl.BlockSpec((1, d), lambda i: (0, 0)),
        ],
        out_specs=pl.BlockSpec((nb, oc), lambda i: (i, 0)),
        out_shape=jax.ShapeDtypeStruct((npad, oc), F32),
    )(z2, z2, h2, cnt, b)


def kernel(x, edge_index, p, cW1, cb1, cW2, cb2, fW1, fb1, fW2, fb2,
           wW1, wb1, wW2, wb2, wW3, wb3, gW0, gb0, gW1, gb1, gW2, gb2):
    n, d = x.shape
    e = edge_index.shape[1]
    npad = ((n + 1023) // 1024) * 1024
    src, dst = edge_index[0], edge_index[1]

    x_pad = jnp.pad(x, ((0, npad - n), (0, 0)))
    w1cat = jnp.concatenate([cW1, fW1[0], fW1[1], fW1[2]], axis=1)
    b1cat = jnp.concatenate([cb1, fb1[0], fb1[1], fb1[2]])[None, :]
    w2bd = jnp.zeros((80, 4), F32)
    w2bd = w2bd.at[0:20, 0].set(cW2[:, 0])
    w2bd = w2bd.at[20:40, 1].set(fW2[0][:, 0])
    w2bd = w2bd.at[40:60, 2].set(fW2[1][:, 0])
    w2bd = w2bd.at[60:80, 3].set(fW2[2][:, 0])
    b2cat = jnp.concatenate([cb2, fb2[0], fb2[1], fb2[2]])[None, :]
    nodeo = _node_mlps(x_pad, w1cat, b1cat, w2bd, b2cat)
    kappa = nodeo[:n, 0]
    fs = [nodeo[:n, 1 + t] for t in range(3)]

    cs = _colsum(wW1)
    w = _edge_weights(cs, wb1[None, :], wW2, wb2[None, :], wW3, wb3[None, :])[0]

    ep = 32 * 80 * 128
    epad = ep - e
    eids_pad = jnp.arange(epad, dtype=jnp.int32)
    trash_pad = n + (eids_pad & 127)
    srcp = jnp.concatenate([src, jnp.zeros((epad,), jnp.int32)])
    dst0 = jnp.concatenate([dst, trash_pad])
    wp = jnp.concatenate([w, jnp.zeros((epad,), F32)])
    src2d = srcp.reshape(ep // 64, 64)
    zeros_z = jnp.zeros((npad, d), F32)
    zeros_n = jnp.zeros((npad,), F32)

    f3 = jnp.pad(jnp.stack(fs), ((0, 0), (0, npad - n)))
    outA = _sc_pass_a(srcp, dst0, wp, f3.reshape(3 * npad), zeros_n)
    sumA = _sum_tiles(outA.reshape(32, 6 * npad)).reshape(6, npad)
    df3 = sumA[0:3]
    gm3 = 0.5 * sumA[3:6]
    g2s = []
    for t in range(3):
        outB = _sc_pass_b(srcp, dst0, wp, f3[t], df3[t], gm3[t], zeros_n)
        sumB = _sum_tiles(outB.reshape(32, 2 * npad)).reshape(2, npad)
        g2s.append(0.5 * (sumB[0] - sumB[1]))

    krows = jnp.pad(kappa[:, None], ((0, npad - n), (0, 0)))
    kcols = jnp.pad(kappa[None, :], ((0, 0), (0, npad - n)),
                    constant_values=-1.0)
    rows = [kcols, gm3[0][None, :], gm3[1][None, :], gm3[2][None, :]]
    rows += [g2[None, :] for g2 in g2s]
    rows += [jnp.zeros((1, npad), F32)]
    cols_pack = jnp.concatenate(rows, axis=0)
    loss, rank = _nsq_pass(krows, cols_pack, n)
    curv_loss = loss[0, 0]
    rankf = rank[:, 0]

    pf = jnp.asarray(p, F32)
    num1 = jnp.floor(pf * 1 * n / 100.0)
    num2 = jnp.floor(pf * 2 * n / 100.0)
    r1 = (rankf < num1).astype(F32)
    r2 = (rankf < num2).astype(F32)
    cnt_o, dstm1, dstm2 = _sc_deg_mask(srcp, dst0, r1, r2, zeros_n, n)
    cnt3 = _sum_tiles(cnt_o.reshape(32, 3 * npad)).reshape(3, npad)
    cnts = [cnt3[0][:, None], cnt3[1][:, None], cnt3[2][:, None]]
    dst2ds = [dst0.reshape(ep // 64, 64), dstm1.reshape(ep // 64, 64),
              dstm2.reshape(ep // 64, 64)]

    hx = x_pad
    Ws = [(gW0, gb0), (gW1, gb1), (gW2, gb2)]
    out = None
    for i in range(3):
        h2 = _mm_scale(hx, Ws[i][0], cnts[i])
        z2 = _sc_aggregate(h2, src2d, dst2ds[i], zeros_z)
        res = _epilogue(z2, h2, cnts[i], Ws[i][1][None, :], final=(i == 2))
        if i < 2:
            hx = res
        else:
            out = res[:n]

    return (out, curv_loss)


# speedup vs baseline: 1.0304x; 1.0304x over previous
import functools

import jax
import jax.numpy as jnp
from jax import lax
from jax.experimental import pallas as pl
from jax.experimental.pallas import tpu as pltpu
from jax.experimental.pallas import tpu_sc as plsc

F32 = jnp.float32
I32 = jnp.int32


def _sc_agg_body(chunks, h2_hbm, src2d_hbm, dst2d_hbm, zeros_hbm,
                 z2_hbm, idx_s, idx_d, r0, r1, g0, g1, zsp):
    c = lax.axis_index("c")
    s = lax.axis_index("s")
    wid = s * 2 + c
    rows = (r0, r1)
    semg = (g0, g1)
    half = chunks // 2

    nrows = zsp.shape[0] // 16
    pltpu.sync_copy(zeros_hbm.at[pl.ds(s * nrows, nrows)],
                    zsp.at[pl.ds(s * nrows, nrows)])
    plsc.subcore_barrier()

    for h in range(2):
        pltpu.sync_copy(
            src2d_hbm.at[pl.ds(wid * chunks + h * half, half)], idx_s)
        pltpu.sync_copy(
            dst2d_hbm.at[pl.ds(wid * chunks + h * half, half)], idx_d)
        pltpu.async_copy(h2_hbm.at[idx_s.at[0]], rows[0], semg[0])

        def body(g, carry):
            for b in range(2):
                ch = g * 2 + b
                pltpu.make_async_copy(h2_hbm.at[idx_s.at[ch]], rows[b],
                                      semg[b]).wait()

                @pl.when(ch < half - 1)
                def _():
                    pltpu.async_copy(h2_hbm.at[idx_s.at[ch + 1]],
                                     rows[1 - b], semg[1 - b])

                pltpu.sync_copy(rows[b], zsp.at[idx_d.at[ch]], add=True)
            return carry

        lax.fori_loop(0, half // 2, body, 0)

    plsc.subcore_barrier()
    pltpu.sync_copy(zsp.at[pl.ds(s * nrows, nrows)],
                    z2_hbm.at[c].at[pl.ds(s * nrows, nrows)])


def _sc_aggregate(h2, src2d, dst2d, zeros_z):
    npad, d = h2.shape
    nchunks = src2d.shape[0] // 32
    assert nchunks % 4 == 0
    mesh = plsc.VectorSubcoreMesh(core_axis_name="c", subcore_axis_name="s")
    return pl.kernel(
        functools.partial(_sc_agg_body, nchunks),
        out_type=jax.ShapeDtypeStruct((2, npad, d), F32),
        mesh=mesh,
        scratch_types=[
            pltpu.VMEM((nchunks // 2, 128), I32),
            pltpu.VMEM((nchunks // 2, 128), I32),
        ] + [pltpu.VMEM((128, d), F32)] * 2
          + [pltpu.SemaphoreType.DMA] * 2
          + [pltpu.VMEM_SHARED((npad, d), F32)],
    )(h2, src2d, dst2d, zeros_z)


def _sc_passa_body(et, src_hbm, dst_hbm, w_hbm, f3_hbm, zeros_hbm,
                   out_hbm, src_c, dst_c, w_c,
                   f0_v, f1_v, f2_v, df0, df1, df2, gm0, gm1, gm2):
    c = lax.axis_index("c")
    s = lax.axis_index("s")
    wid = s * 2 + c
    base = wid * et
    f_vs = (f0_v, f1_v, f2_v)
    df_vs = (df0, df1, df2)
    gm_vs = (gm0, gm1, gm2)
    npad = f0_v.shape[0]
    for t in range(3):
        pltpu.sync_copy(f3_hbm.at[pl.ds(t * npad, npad)], f_vs[t])
        pltpu.sync_copy(zeros_hbm, df_vs[t])
        pltpu.sync_copy(zeros_hbm, gm_vs[t])

    def outer(o, carry):
        off = base + o * 1024
        pltpu.sync_copy(src_hbm.at[pl.ds(off, 1024)], src_c)
        pltpu.sync_copy(dst_hbm.at[pl.ds(off, 1024)], dst_c)
        pltpu.sync_copy(w_hbm.at[pl.ds(off, 1024)], w_c)

        def inner(k, carry2):
            s16 = src_c[pl.ds(k * 16, 16)]
            d16 = dst_c[pl.ds(k * 16, 16)]
            w16 = w_c[pl.ds(k * 16, 16)]
            for t in range(3):
                fs = plsc.load_gather(f_vs[t], [s16])
                fd = plsc.load_gather(f_vs[t], [d16]) - fs
                wfd = w16 * fd
                plsc.addupdate_scatter(df_vs[t], [s16], wfd)
                plsc.addupdate_scatter(gm_vs[t], [s16], wfd * fd)
            return carry2

        return lax.fori_loop(0, 64, inner, carry)

    lax.fori_loop(0, et // 1024, outer, 0)
    for t in range(3):
        pltpu.sync_copy(df_vs[t], out_hbm.at[pl.ds((wid * 6 + t) * npad, npad)])
        pltpu.sync_copy(gm_vs[t],
                        out_hbm.at[pl.ds((wid * 6 + 3 + t) * npad, npad)])


def _sc_pass_a(srcp, dstp, wp, f3, zeros_n):
    npad = f3.shape[0] // 3
    et = srcp.shape[0] // 32
    mesh = plsc.VectorSubcoreMesh(core_axis_name="c", subcore_axis_name="s")
    return pl.kernel(
        functools.partial(_sc_passa_body, et),
        out_type=jax.ShapeDtypeStruct((32 * 6 * npad,), F32),
        mesh=mesh,
        compiler_params=pltpu.CompilerParams(needs_layout_passes=False),
        scratch_types=[
            pltpu.VMEM((1024,), I32),
            pltpu.VMEM((1024,), I32),
            pltpu.VMEM((1024,), F32),
        ] + [pltpu.VMEM((npad,), F32)] * 9,
    )(srcp, dstp, wp, f3, zeros_n)


def _sc_passb_body(et, src_hbm, dst_hbm, w_hbm, f_hbm, df_hbm, gm_hbm,
                   zeros_hbm, out_hbm,
                   src_c, dst_c, w_c, f_v, df_v, gm_v, dg_v, gf_v):
    c = lax.axis_index("c")
    s = lax.axis_index("s")
    wid = s * 2 + c
    base = wid * et
    pltpu.sync_copy(f_hbm, f_v)
    pltpu.sync_copy(df_hbm, df_v)
    pltpu.sync_copy(gm_hbm, gm_v)
    pltpu.sync_copy(zeros_hbm, dg_v)
    pltpu.sync_copy(zeros_hbm, gf_v)

    def outer(o, carry):
        off = base + o * 1024
        pltpu.sync_copy(src_hbm.at[pl.ds(off, 1024)], src_c)
        pltpu.sync_copy(dst_hbm.at[pl.ds(off, 1024)], dst_c)
        pltpu.sync_copy(w_hbm.at[pl.ds(off, 1024)], w_c)

        def inner(k, carry2):
            s16 = src_c[pl.ds(k * 16, 16)]
            d16 = dst_c[pl.ds(k * 16, 16)]
            w16 = w_c[pl.ds(k * 16, 16)]
            fd = plsc.load_gather(f_v, [d16]) - plsc.load_gather(f_v, [s16])
            gd = plsc.load_gather(gm_v, [d16]) - plsc.load_gather(gm_v, [s16])
            dfd = plsc.load_gather(df_v, [d16]) - plsc.load_gather(df_v, [s16])
            plsc.addupdate_scatter(dg_v, [s16], w16 * gd)
            plsc.addupdate_scatter(gf_v, [s16], w16 * fd * dfd)
            return carry2

        return lax.fori_loop(0, 64, inner, carry)

    lax.fori_loop(0, et // 1024, outer, 0)
    npad = dg_v.shape[0]
    pltpu.sync_copy(dg_v, out_hbm.at[pl.ds((wid * 2 + 0) * npad, npad)])
    pltpu.sync_copy(gf_v, out_hbm.at[pl.ds((wid * 2 + 1) * npad, npad)])


def _sc_pass_b(srcp, dstp, wp, f, df, gm, zeros_n):
    npad = f.shape[0]
    et = srcp.shape[0] // 32
    mesh = plsc.VectorSubcoreMesh(core_axis_name="c", subcore_axis_name="s")
    return pl.kernel(
        functools.partial(_sc_passb_body, et),
        out_type=jax.ShapeDtypeStruct((32 * 2 * npad,), F32),
        mesh=mesh,
        compiler_params=pltpu.CompilerParams(needs_layout_passes=False),
        scratch_types=[
            pltpu.VMEM((1024,), I32),
            pltpu.VMEM((1024,), I32),
            pltpu.VMEM((1024,), F32),
        ] + [pltpu.VMEM((npad,), F32)] * 5,
    )(srcp, dstp, wp, f, df, gm, zeros_n)


def _sc_deg_body(et, n_real, src_hbm, dst_hbm, r1_hbm, r2_hbm, zeros_hbm,
                 cnt_hbm, dstm1_hbm, dstm2_hbm,
                 src_c, dst_c, m1_c, m2_c, r1_v, r2_v, dg0, dg1, dg2):
    c = lax.axis_index("c")
    s = lax.axis_index("s")
    wid = s * 2 + c
    base = wid * et
    pltpu.sync_copy(r1_hbm, r1_v)
    pltpu.sync_copy(r2_hbm, r2_v)
    pltpu.sync_copy(zeros_hbm, dg0)
    pltpu.sync_copy(zeros_hbm, dg1)
    pltpu.sync_copy(zeros_hbm, dg2)
    ones16 = jnp.ones((16,), F32)
    lane16 = lax.broadcasted_iota(I32, (16,), 0)

    def outer(o, carry):
        off = base + o * 1024
        pltpu.sync_copy(src_hbm.at[pl.ds(off, 1024)], src_c)
        pltpu.sync_copy(dst_hbm.at[pl.ds(off, 1024)], dst_c)

        def inner(k, carry2):
            s16 = src_c[pl.ds(k * 16, 16)]
            d16 = dst_c[pl.ds(k * 16, 16)]
            r1s = plsc.load_gather(r1_v, [s16])
            r1d = plsc.load_gather(r1_v, [d16])
            r2s = plsc.load_gather(r2_v, [s16])
            r2d = plsc.load_gather(r2_v, [d16])
            keep1 = (r1s == 0.0) & (r1d == 0.0)
            keep2 = (r2s == 0.0) & (r2d == 0.0)
            plsc.addupdate_scatter(dg0, [d16], ones16)
            plsc.addupdate_scatter(dg1, [d16], jnp.where(keep1, 1.0, 0.0))
            plsc.addupdate_scatter(dg2, [d16], jnp.where(keep2, 1.0, 0.0))
            trash16 = n_real + ((off + k * 16 + lane16) & 127)
            m1_c[pl.ds(k * 16, 16)] = jnp.where(keep1, d16, trash16)
            m2_c[pl.ds(k * 16, 16)] = jnp.where(keep2, d16, trash16)
            return carry2

        lax.fori_loop(0, 64, inner, carry)
        pltpu.sync_copy(m1_c, dstm1_hbm.at[pl.ds(off, 1024)])
        pltpu.sync_copy(m2_c, dstm2_hbm.at[pl.ds(off, 1024)])
        return carry

    lax.fori_loop(0, et // 1024, outer, 0)
    npad = dg0.shape[0]
    pltpu.sync_copy(dg0, cnt_hbm.at[pl.ds((wid * 3 + 0) * npad, npad)])
    pltpu.sync_copy(dg1, cnt_hbm.at[pl.ds((wid * 3 + 1) * npad, npad)])
    pltpu.sync_copy(dg2, cnt_hbm.at[pl.ds((wid * 3 + 2) * npad, npad)])


def _sc_deg_mask(srcp, dst0, r1, r2, zeros_n, n_real):
    ep = srcp.shape[0]
    et = ep // 32
    npad = r1.shape[0]
    mesh = plsc.VectorSubcoreMesh(core_axis_name="c", subcore_axis_name="s")
    return pl.kernel(
        functools.partial(_sc_deg_body, et, n_real),
        compiler_params=pltpu.CompilerParams(needs_layout_passes=False),
        out_type=[
            jax.ShapeDtypeStruct((32 * 3 * npad,), F32),
            jax.ShapeDtypeStruct((ep,), I32),
            jax.ShapeDtypeStruct((ep,), I32),
        ],
        mesh=mesh,
        scratch_types=[
            pltpu.VMEM((1024,), I32),
            pltpu.VMEM((1024,), I32),
            pltpu.VMEM((1024,), I32),
            pltpu.VMEM((1024,), I32),
            pltpu.VMEM((npad,), F32),
            pltpu.VMEM((npad,), F32),
            pltpu.VMEM((npad,), F32),
            pltpu.VMEM((npad,), F32),
            pltpu.VMEM((npad,), F32),
        ],
    )(srcp, dst0, r1, r2, zeros_n)


def _sumtiles_body(x_ref, o_ref):
    o_ref[...] = jnp.sum(x_ref[...], axis=0, keepdims=True)


def _sum_tiles(x, cb=2048):
    t, m = x.shape
    return pl.pallas_call(
        _sumtiles_body,
        grid=(m // cb,),
        in_specs=[pl.BlockSpec((t, cb), lambda i: (0, i))],
        out_specs=pl.BlockSpec((1, cb), lambda i: (0, i)),
        out_shape=jax.ShapeDtypeStruct((1, m), F32),
    )(x)


def _node_mlp_body(x_ref, w1_ref, b1_ref, w2_ref, b2_ref, o_ref):
    h = jnp.maximum(
        jnp.dot(x_ref[...], w1_ref[...], preferred_element_type=F32) + b1_ref[...],
        0.0)
    o_ref[...] = jax.nn.sigmoid(
        jnp.dot(h, w2_ref[...], preferred_element_type=F32) + b2_ref[...])


def _node_mlps(x_pad, w1cat, b1cat, w2bd, b2cat, nb=1024):
    npad = x_pad.shape[0]
    d = x_pad.shape[1]
    k = w1cat.shape[1]
    grid = (npad // nb,)
    return pl.pallas_call(
        _node_mlp_body,
        grid=grid,
        in_specs=[
            pl.BlockSpec((nb, d), lambda i: (i, 0)),
            pl.BlockSpec((d, k), lambda i: (0, 0)),
            pl.BlockSpec((1, k), lambda i: (0, 0)),
            pl.BlockSpec((k, 4), lambda i: (0, 0)),
            pl.BlockSpec((1, 4), lambda i: (0, 0)),
        ],
        out_specs=pl.BlockSpec((nb, 4), lambda i: (i, 0)),
        out_shape=jax.ShapeDtypeStruct((npad, 4), F32),
    )(x_pad, w1cat, b1cat, w2bd, b2cat)


def _colsum_body(w_ref, o_ref):
    part = jnp.sum(w_ref[...], axis=0, keepdims=True)

    @pl.when(pl.program_id(0) == 0)
    def _():
        o_ref[...] = part

    @pl.when(pl.program_id(0) != 0)
    def _():
        o_ref[...] = o_ref[...] + part


def _colsum(w1, eb=3200):
    e, c = w1.shape
    return pl.pallas_call(
        _colsum_body,
        grid=(e // eb,),
        in_specs=[pl.BlockSpec((eb, c), lambda i: (i, 0))],
        out_specs=pl.BlockSpec((1, c), lambda i: (0, 0)),
        out_shape=jax.ShapeDtypeStruct((1, c), F32),
    )(w1)


def _wmlp_body(cs_ref, b1_ref, w2_ref, b2_ref, w3_ref, b3_ref, o_ref, h2_ref):
    @pl.when(pl.program_id(0) == 0)
    def _():
        h1 = jnp.maximum(cs_ref[...] + b1_ref[...], 0.0)
        h2_ref[...] = jnp.maximum(
            jnp.dot(h1, w2_ref[...], preferred_element_type=F32) + b2_ref[...], 0.0)

    o_ref[...] = jax.nn.sigmoid(
        jnp.dot(h2_ref[...], w3_ref[...], preferred_element_type=F32) + b3_ref[...])


def _edge_weights(cs, wb1, wW2, wb2, wW3, wb3, cb=3200):
    e = wW3.shape[1]
    return pl.pallas_call(
        _wmlp_body,
        grid=(e // cb,),
        in_specs=[
            pl.BlockSpec((1, 64), lambda i: (0, 0)),
            pl.BlockSpec((1, 64), lambda i: (0, 0)),
            pl.BlockSpec((64, 64), lambda i: (0, 0)),
            pl.BlockSpec((1, 64), lambda i: (0, 0)),
            pl.BlockSpec((64, cb), lambda i: (0, i)),
            pl.BlockSpec((1, cb), lambda i: (0, i)),
        ],
        out_specs=pl.BlockSpec((1, cb), lambda i: (0, i)),
        out_shape=jax.ShapeDtypeStruct((1, e), F32),
        scratch_shapes=[pltpu.VMEM((1, 64), F32)],
    )(cs, wb1, wW2, wb2, wW3, wb3)


def _nsq_body(n_real, rb, cb, krow_ref, pack_ref, loss_ref, rank_ref):
    i = pl.program_id(0)
    j = pl.program_id(1)
    krow = krow_ref[...]
    pack = pack_ref[...]
    kcol = pack[0:1, :]
    row_gid = jax.lax.broadcasted_iota(jnp.int32, (rb, 1), 0) + i * rb
    col_gid = jax.lax.broadcasted_iota(jnp.int32, (1, cb), 1) + j * cb
    rowvalid = (row_gid < n_real).astype(F32)

    acc = jnp.maximum(krow * pack[1:2, :] - pack[4:5, :], 0.0)
    acc = acc + jnp.maximum(krow * pack[2:3, :] - pack[5:6, :], 0.0)
    acc = acc + jnp.maximum(krow * pack[3:4, :] - pack[6:7, :], 0.0)
    loss_tile = jnp.sum(jnp.sum(acc, axis=1, keepdims=True) * rowvalid)

    gt = (kcol > krow).astype(F32)
    eqlow = jnp.where((kcol == krow) & (col_gid < row_gid), 1.0, 0.0)
    rblk = jnp.sum(gt + eqlow, axis=1, keepdims=True)

    @pl.when((i == 0) & (j == 0))
    def _():
        loss_ref[...] = jnp.zeros((1, 1), F32)

    @pl.when(j == 0)
    def _():
        loss_ref[...] += jnp.full((1, 1), -3.0) * jnp.sum(krow * rowvalid)
        rank_ref[...] = rblk

    @pl.when(j != 0)
    def _():
        rank_ref[...] = rank_ref[...] + rblk

    loss_ref[...] += loss_tile.reshape(1, 1)


def _nsq_pass(krows, cols_pack, n_real, rb=256, cb=1024):
    npad = krows.shape[0]
    grid = (npad // rb, npad // cb)
    return pl.pallas_call(
        functools.partial(_nsq_body, n_real, rb, cb),
        grid=grid,
        in_specs=[
            pl.BlockSpec((rb, 1), lambda i, j: (i, 0)),
            pl.BlockSpec((8, cb), lambda i, j: (0, j)),
        ],
        out_specs=[
            pl.BlockSpec((1, 1), lambda i, j: (0, 0)),
            pl.BlockSpec((rb, 1), lambda i, j: (i, 0)),
        ],
        out_shape=[
            jax.ShapeDtypeStruct((1, 1), F32),
            jax.ShapeDtypeStruct((npad, 1), F32),
        ],
    )(krows, cols_pack)


def _mm_scale_body(hx_ref, w_ref, cnt_ref, o_ref):
    dis = lax.rsqrt(cnt_ref[...] + 1.0)
    o_ref[...] = dis * jnp.dot(
        hx_ref[...], w_ref[...], preferred_element_type=F32)


def _mm_scale(hx, w, cnt, nb=1024):
    npad, d = hx.shape
    o = w.shape[1]
    return pl.pallas_call(
        _mm_scale_body,
        grid=(npad // nb,),
        in_specs=[
            pl.BlockSpec((nb, d), lambda i: (i, 0)),
            pl.BlockSpec((d, o), lambda i: (0, 0)),
            pl.BlockSpec((nb, 1), lambda i: (i, 0)),
        ],
        out_specs=pl.BlockSpec((nb, o), lambda i: (i, 0)),
        out_shape=jax.ShapeDtypeStruct((npad, o), F32),
    )(hx, w, cnt)


def _epi_body(final, za_ref, zb_ref, h2_ref, cnt_ref, b_ref, o_ref):
    dis = lax.rsqrt(cnt_ref[...] + 1.0)
    t = dis * (za_ref[0] + zb_ref[0] + h2_ref[...]) + b_ref[...]
    if final:
        s = jax.nn.sigmoid(t)
        o_ref[...] = jnp.sum(s, axis=1, keepdims=True) * (1.0 / s.shape[1])
    else:
        o_ref[...] = jnp.maximum(t, 0.0)


def _epilogue(z2, h2, cnt, b, final, nb=1024):
    _, npad, d = z2.shape
    oc = 1 if final else d
    return pl.pallas_call(
        functools.partial(_epi_body, final),
        grid=(npad // nb,),
        in_specs=[
            pl.BlockSpec((1, nb, d), lambda i: (0, i, 0)),
            pl.BlockSpec((1, nb, d), lambda i: (1, i, 0)),
            pl.BlockSpec((nb, d), lambda i: (i, 0)),
            pl.BlockSpec((nb, 1), lambda i: (i, 0)),
            pl.BlockSpec((1, d), lambda i: (0, 0)),
        ],
        out_specs=pl.BlockSpec((nb, oc), lambda i: (i, 0)),
        out_shape=jax.ShapeDtypeStruct((npad, oc), F32),
    )(z2, z2, h2, cnt, b)


def kernel(x, edge_index, p, cW1, cb1, cW2, cb2, fW1, fb1, fW2, fb2,
           wW1, wb1, wW2, wb2, wW3, wb3, gW0, gb0, gW1, gb1, gW2, gb2):
    n, d = x.shape
    e = edge_index.shape[1]
    npad = ((n + 1023) // 1024) * 1024
    src, dst = edge_index[0], edge_index[1]

    x_pad = jnp.pad(x, ((0, npad - n), (0, 0)))
    w1cat = jnp.concatenate([cW1, fW1[0], fW1[1], fW1[2]], axis=1)
    b1cat = jnp.concatenate([cb1, fb1[0], fb1[1], fb1[2]])[None, :]
    w2bd = jnp.zeros((80, 4), F32)
    w2bd = w2bd.at[0:20, 0].set(cW2[:, 0])
    w2bd = w2bd.at[20:40, 1].set(fW2[0][:, 0])
    w2bd = w2bd.at[40:60, 2].set(fW2[1][:, 0])
    w2bd = w2bd.at[60:80, 3].set(fW2[2][:, 0])
    b2cat = jnp.concatenate([cb2, fb2[0], fb2[1], fb2[2]])[None, :]
    nodeo = _node_mlps(x_pad, w1cat, b1cat, w2bd, b2cat)
    kappa = nodeo[:n, 0]
    fs = [nodeo[:n, 1 + t] for t in range(3)]

    cs = _colsum(wW1)
    w = _edge_weights(cs, wb1[None, :], wW2, wb2[None, :], wW3, wb3[None, :])[0]

    ep = 32 * 80 * 128
    epad = ep - e
    eids_pad = jnp.arange(epad, dtype=jnp.int32)
    trash_pad = n + (eids_pad & 127)
    srcp = jnp.concatenate([src, jnp.zeros((epad,), jnp.int32)])
    dst0 = jnp.concatenate([dst, trash_pad])
    wp = jnp.concatenate([w, jnp.zeros((epad,), F32)])
    src2d = srcp.reshape(ep // 128, 128)
    zeros_z = jnp.zeros((npad, d), F32)
    zeros_n = jnp.zeros((npad,), F32)

    f3 = jnp.pad(jnp.stack(fs), ((0, 0), (0, npad - n)))
    outA = _sc_pass_a(srcp, dst0, wp, f3.reshape(3 * npad), zeros_n)
    sumA = _sum_tiles(outA.reshape(32, 6 * npad)).reshape(6, npad)
    df3 = sumA[0:3]
    gm3 = 0.5 * sumA[3:6]
    g2s = []
    for t in range(3):
        outB = _sc_pass_b(srcp, dst0, wp, f3[t], df3[t], gm3[t], zeros_n)
        sumB = _sum_tiles(outB.reshape(32, 2 * npad)).reshape(2, npad)
        g2s.append(0.5 * (sumB[0] - sumB[1]))

    krows = jnp.pad(kappa[:, None], ((0, npad - n), (0, 0)))
    kcols = jnp.pad(kappa[None, :], ((0, 0), (0, npad - n)),
                    constant_values=-1.0)
    rows = [kcols, gm3[0][None, :], gm3[1][None, :], gm3[2][None, :]]
    rows += [g2[None, :] for g2 in g2s]
    rows += [jnp.zeros((1, npad), F32)]
    cols_pack = jnp.concatenate(rows, axis=0)
    loss, rank = _nsq_pass(krows, cols_pack, n)
    curv_loss = loss[0, 0]
    rankf = rank[:, 0]

    pf = jnp.asarray(p, F32)
    num1 = jnp.floor(pf * 1 * n / 100.0)
    num2 = jnp.floor(pf * 2 * n / 100.0)
    r1 = (rankf < num1).astype(F32)
    r2 = (rankf < num2).astype(F32)
    cnt_o, dstm1, dstm2 = _sc_deg_mask(srcp, dst0, r1, r2, zeros_n, n)
    cnt3 = _sum_tiles(cnt_o.reshape(32, 3 * npad)).reshape(3, npad)
    cnts = [cnt3[0][:, None], cnt3[1][:, None], cnt3[2][:, None]]
    dst2ds = [dst0.reshape(ep // 128, 128), dstm1.reshape(ep // 128, 128),
              dstm2.reshape(ep // 128, 128)]

    hx = x_pad
    Ws = [(gW0, gb0), (gW1, gb1), (gW2, gb2)]
    out = None
    for i in range(3):
        h2 = _mm_scale(hx, Ws[i][0], cnts[i])
        z2 = _sc_aggregate(h2, src2d, dst2ds[i], zeros_z)
        res = _epilogue(z2, h2, cnts[i], Ws[i][1][None, :], final=(i == 2))
        if i < 2:
            hx = res
        else:
            out = res[:n]

    return (out, curv_loss)
